# baseline (speedup 1.0000x reference)
import functools

import jax
import jax.numpy as jnp
from jax import lax
from jax.experimental import pallas as pl
from jax.experimental.pallas import tpu as pltpu
from jax.experimental.pallas import tpu_sc as plsc

CROP_H = 480
CROP_W = 480
NUM_CROPS = 2

_NUM_CORES = 2
_NUM_SUBCORES = 16
_NW = _NUM_CORES * _NUM_SUBCORES

_R_CHUNK = 80
_CHUNKS_PER_PLANE = CROP_H // _R_CHUNK
_NBUF = 3
_GROUP = 240
_NSPLIT = 3


def _crop_offsets(B, H, W):
    k = jax.random.key(1)
    kh, kw = jax.random.split(k)
    ih = ((H - CROP_H) * jax.random.uniform(kh, (B, NUM_CROPS))).astype(jnp.int32)
    iw = ((W - CROP_W) * jax.random.uniform(kw, (B, NUM_CROPS))).astype(jnp.int32)
    return ih, iw


def _make_crop_kernel(planes_per_w, W):
    n_chunks = planes_per_w * _CHUNKS_PER_PLANE
    mesh = plsc.VectorSubcoreMesh(core_axis_name="c", subcore_axis_name="s")

    @functools.partial(
        pl.kernel,
        out_type=jax.ShapeDtypeStruct(
            (_NW * planes_per_w * CROP_H, CROP_W), jnp.float32
        ),
        mesh=mesh,
        compiler_params=pltpu.CompilerParams(
            use_tc_tiling_on_sc=False, needs_layout_passes=False
        ),
        scratch_types=[
            pltpu.VMEM((planes_per_w, 16), jnp.int32),
            pltpu.VMEM((planes_per_w, 16), jnp.int32),
        ]
        + [pltpu.VMEM((_R_CHUNK,), jnp.int32)] * _NBUF
        + [pltpu.VMEM((_R_CHUNK, W), jnp.float32)] * _NBUF
        + [pltpu.SemaphoreType.DMA] * (2 * _NBUF),
    )
    def _crop_copy(in_hbm, rs_hbm, cs_hbm, out_hbm, rs_v, cs_v, *scratch):
        idx = scratch[0:_NBUF]
        buf = scratch[_NBUF : 2 * _NBUF]
        gsem = scratch[2 * _NBUF : 3 * _NBUF]
        wsem = scratch[3 * _NBUF : 4 * _NBUF]
        wid = lax.axis_index("s") * _NUM_CORES + lax.axis_index("c")
        pltpu.sync_copy(rs_hbm.at[wid], rs_v)
        pltpu.sync_copy(cs_hbm.at[wid], cs_v)
        iota = lax.iota(jnp.int32, 16)

        def build_idx(g):
            slot, ci = divmod(g, _CHUNKS_PER_PLANE)
            rs_vec = rs_v[slot]
            r0 = ci * _R_CHUNK
            for k in range(0, _R_CHUNK, 16):
                idx[g % _NBUF][pl.ds(k, 16)] = rs_vec + (r0 + k) + iota

        def start_gather(g):
            pltpu.async_copy(in_hbm.at[idx[g % _NBUF]], buf[g % _NBUF], gsem[g % _NBUF])

        def wait_gather(g):
            pltpu.make_async_copy(
                in_hbm.at[idx[g % _NBUF]], buf[g % _NBUF], gsem[g % _NBUF]
            ).wait()

        def write_args(g):
            slot, ci = divmod(g, _CHUNKS_PER_PLANE)
            dst = (wid * planes_per_w + slot) * CROP_H + ci * _R_CHUNK
            dst = pl.multiple_of(dst, 8)
            return (
                buf[g % _NBUF].at[:, pl.ds(0, CROP_W)],
                out_hbm.at[pl.ds(dst, _R_CHUNK)],
                wsem[g % _NBUF],
            )

        def shift(g):
            slot = g // _CHUNKS_PER_PLANE
            cs = jnp.max(cs_v[slot])
            b = buf[g % _NBUF]

            def _row(i, carry):
                for k0 in range(0, CROP_W, _GROUP):
                    vals = [
                        b[i, pl.ds(cs + k0 + k, 16)] for k in range(0, _GROUP, 16)
                    ]
                    for k, v in zip(range(0, _GROUP, 16), vals):
                        b[i, pl.ds(k0 + k, 16)] = v
                return carry

            lax.fori_loop(0, _R_CHUNK, _row, 0)

        build_idx(0)
        start_gather(0)
        for g in range(n_chunks):
            if g + 1 < n_chunks:
                if g >= 2:
                    pltpu.make_async_copy(*write_args(g - 2)).wait()
                build_idx(g + 1)
                start_gather(g + 1)
            wait_gather(g)
            shift(g)
            pltpu.async_copy(*write_args(g))
        for j in range(max(0, n_chunks - 3), n_chunks):
            pltpu.make_async_copy(*write_args(j)).wait()

    return _crop_copy


def kernel(inputs):
    B, C, H, W = inputs.shape
    ih, iw = _crop_offsets(B, H, W)

    P = B * NUM_CROPS * C
    p = jnp.arange(P)
    b_idx = p // (NUM_CROPS * C)
    n_idx = (p // C) % NUM_CROPS
    c_idx = p % C
    row_start = (b_idx * C + c_idx) * H + ih[b_idx, n_idx]
    col_start = iw[b_idx, n_idx]

    per_split = P // _NSPLIT
    planes_per_w = per_split // _NW
    in2d = inputs.reshape(B * C * H, W)
    crop_call = _make_crop_kernel(planes_per_w, W)

    outs = []
    for s in range(_NSPLIT):
        sl = slice(s * per_split, (s + 1) * per_split)
        rs_rep = jnp.broadcast_to(
            row_start[sl].reshape(_NW, planes_per_w, 1).astype(jnp.int32),
            (_NW, planes_per_w, 16),
        )
        cs_rep = jnp.broadcast_to(
            col_start[sl].reshape(_NW, planes_per_w, 1).astype(jnp.int32),
            (_NW, planes_per_w, 16),
        )
        outs.append(crop_call(in2d, rs_rep, cs_rep))
    out2d = jnp.concatenate(outs, axis=0)
    return out2d.reshape(B * NUM_CROPS, C, CROP_H, CROP_W)


# speedup vs baseline: 2.6629x; 2.6629x over previous
import functools

import jax
import jax.numpy as jnp
from jax import lax
from jax.experimental import pallas as pl
from jax.experimental.pallas import tpu as pltpu
from jax.experimental.pallas import tpu_sc as plsc

CROP_H = 480
CROP_W = 480
NUM_CROPS = 2

_NUM_CORES = 2
_NUM_SUBCORES = 16
_NW = _NUM_CORES * _NUM_SUBCORES

_R_CHUNK = 48
_CHUNKS_PER_PLANE = CROP_H // _R_CHUNK


def _crop_offsets(B, H, W):
    k = jax.random.key(1)
    kh, kw = jax.random.split(k)
    ih = ((H - CROP_H) * jax.random.uniform(kh, (B, NUM_CROPS))).astype(jnp.int32)
    iw = ((W - CROP_W) * jax.random.uniform(kw, (B, NUM_CROPS))).astype(jnp.int32)
    return ih, iw


def kernel(inputs):
    B, C, H, W = inputs.shape
    ih, iw = _crop_offsets(B, H, W)

    P = B * NUM_CROPS * C
    p = jnp.arange(P)
    b_idx = p // (NUM_CROPS * C)
    n_idx = (p // C) % NUM_CROPS
    c_idx = p % C
    row_start = (b_idx * C + c_idx) * H + ih[b_idx, n_idx]
    col_start = iw[b_idx, n_idx]
    planes_per_w = P // _NW
    n_chunks = planes_per_w * _CHUNKS_PER_PLANE
    rs_rep = jnp.broadcast_to(
        row_start.reshape(_NW, planes_per_w, 1).astype(jnp.int32),
        (_NW, planes_per_w, 16),
    )
    cs_rep = jnp.broadcast_to(
        col_start.reshape(_NW, planes_per_w, 1).astype(jnp.int32),
        (_NW, planes_per_w, 16),
    )

    in2d = inputs.reshape(B * C * H, W)
    mesh = plsc.VectorSubcoreMesh(core_axis_name="c", subcore_axis_name="s")

    @functools.partial(
        pl.kernel,
        out_type=jax.ShapeDtypeStruct((P * CROP_H, CROP_W), jnp.float32),
        mesh=mesh,
        compiler_params=pltpu.CompilerParams(needs_layout_passes=False),
        scratch_types=[
            pltpu.VMEM((planes_per_w, 16), jnp.int32),
            pltpu.VMEM((planes_per_w, 16), jnp.int32),
            pltpu.VMEM((_R_CHUNK,), jnp.int32),
            pltpu.VMEM((_R_CHUNK,), jnp.int32),
            pltpu.VMEM((_R_CHUNK, W), jnp.float32),
            pltpu.VMEM((_R_CHUNK, W), jnp.float32),
            pltpu.VMEM((_R_CHUNK, CROP_W), jnp.float32),
            pltpu.VMEM((_R_CHUNK, CROP_W), jnp.float32),
            pltpu.SemaphoreType.DMA,
            pltpu.SemaphoreType.DMA,
            pltpu.SemaphoreType.DMA,
            pltpu.SemaphoreType.DMA,
        ],
    )
    def _crop_copy(
        in_hbm, rs_hbm, cs_hbm, out_hbm,
        rs_v, cs_v, idx0, idx1, ib0, ib1, ob0, ob1, gs0, gs1, ws0, ws1,
    ):
        idx = (idx0, idx1)
        ibuf = (ib0, ib1)
        obuf = (ob0, ob1)
        gsem = (gs0, gs1)
        wsem = (ws0, ws1)
        wid = lax.axis_index("s") * _NUM_CORES + lax.axis_index("c")
        pltpu.sync_copy(rs_hbm.at[wid], rs_v)
        pltpu.sync_copy(cs_hbm.at[wid], cs_v)
        iota = lax.iota(jnp.int32, 16)

        def build_idx(par, c):
            plane = c // _CHUNKS_PER_PLANE
            ci = c % _CHUNKS_PER_PLANE
            rs_vec = rs_v[plane]
            r0 = ci * _R_CHUNK
            for k in range(0, _R_CHUNK, 16):
                idx[par][pl.ds(k, 16)] = rs_vec + (r0 + k) + iota

        def start_gather(par):
            pltpu.async_copy(in_hbm.at[idx[par]], ibuf[par], gsem[par])

        def wait_gather(par):
            pltpu.make_async_copy(in_hbm.at[idx[par]], ibuf[par], gsem[par]).wait()

        def write_args(par, c):
            plane = c // _CHUNKS_PER_PLANE
            ci = c % _CHUNKS_PER_PLANE
            dst = (wid * planes_per_w + plane) * CROP_H + ci * _R_CHUNK
            dst = pl.multiple_of(dst, 16)
            return (obuf[par], out_hbm.at[pl.ds(dst, _R_CHUNK)], wsem[par])

        def shift(par, c):
            plane = c // _CHUNKS_PER_PLANE
            cs = jnp.max(cs_v[plane])
            r_vec = cs_v[plane] & 15
            r = cs & 15
            cs16 = pl.multiple_of(cs - r, 16)
            perm = (iota + r_vec) & 15
            mask = iota < (16 - r_vec)
            src = ibuf[par]
            dst = obuf[par]

            def _row(i, carry):
                vals = [
                    src[i, pl.ds(cs16 + k, 16)] for k in range(0, CROP_W + 16, 16)
                ]
                for k in range(0, CROP_W, 16):
                    lo = vals[k // 16].at[perm].get(mode="promise_in_bounds")
                    hi = vals[k // 16 + 1].at[perm].get(mode="promise_in_bounds")
                    dst[i, pl.ds(k, 16)] = jnp.where(mask, lo, hi)
                return carry

            lax.fori_loop(0, _R_CHUNK, _row, 0)

        def process(par, c, j):
            wait_gather(par)

            @pl.when(j >= 1)
            def _():
                pltpu.make_async_copy(*write_args(par, c)).wait()

            shift(par, c)
            pltpu.async_copy(*write_args(par, c))

        build_idx(0, 0)
        start_gather(0)

        def body(j, carry):
            c0 = 2 * j
            c1 = c0 + 1
            build_idx(1, c1)
            start_gather(1)
            process(0, c0, j)

            @pl.when(j < n_chunks // 2 - 1)
            def _():
                build_idx(0, c0 + 2)
                start_gather(0)

            process(1, c1, j)
            return carry

        lax.fori_loop(0, n_chunks // 2, body, 0)
        pltpu.make_async_copy(*write_args(0, n_chunks - 2)).wait()
        pltpu.make_async_copy(*write_args(1, n_chunks - 1)).wait()

    out2d = _crop_copy(in2d, rs_rep, cs_rep)
    return out2d.reshape(B * NUM_CROPS, C, CROP_H, CROP_W)


# speedup vs baseline: 2.8651x; 1.0759x over previous
import functools

import jax
import jax.numpy as jnp
from jax import lax
from jax.experimental import pallas as pl
from jax.experimental.pallas import tpu as pltpu
from jax.experimental.pallas import tpu_sc as plsc

CROP_H = 480
CROP_W = 480
NUM_CROPS = 2

_NUM_CORES = 2
_NUM_SUBCORES = 16
_NW = _NUM_CORES * _NUM_SUBCORES

_R_CHUNK = 48
_CHUNKS_PER_PLANE = CROP_H // _R_CHUNK


def _crop_offsets(B, H, W):
    k = jax.random.key(1)
    kh, kw = jax.random.split(k)
    ih = ((H - CROP_H) * jax.random.uniform(kh, (B, NUM_CROPS))).astype(jnp.int32)
    iw = ((W - CROP_W) * jax.random.uniform(kw, (B, NUM_CROPS))).astype(jnp.int32)
    return ih, iw


def kernel(inputs):
    B, C, H, W = inputs.shape
    ih, iw = _crop_offsets(B, H, W)

    P = B * NUM_CROPS * C
    p = jnp.arange(P)
    b_idx = p // (NUM_CROPS * C)
    n_idx = (p // C) % NUM_CROPS
    c_idx = p % C
    row_start = (b_idx * C + c_idx) * H + ih[b_idx, n_idx]
    col_start = iw[b_idx, n_idx]
    planes_per_w = P // _NW
    n_chunks = planes_per_w * _CHUNKS_PER_PLANE
    rs_rep = jnp.broadcast_to(
        row_start.reshape(_NW, planes_per_w, 1).astype(jnp.int32),
        (_NW, planes_per_w, 16),
    )
    cs_rep = jnp.broadcast_to(
        col_start.reshape(_NW, planes_per_w, 1).astype(jnp.int32),
        (_NW, planes_per_w, 16),
    )

    in2d = inputs.reshape(B * C * H, W)
    mesh = plsc.VectorSubcoreMesh(core_axis_name="c", subcore_axis_name="s")

    @functools.partial(
        pl.kernel,
        out_type=jax.ShapeDtypeStruct((P * CROP_H, CROP_W), jnp.float32),
        mesh=mesh,
        compiler_params=pltpu.CompilerParams(needs_layout_passes=False),
        scratch_types=[
            pltpu.VMEM((planes_per_w, 16), jnp.int32),
            pltpu.VMEM((planes_per_w, 16), jnp.int32),
            pltpu.VMEM((_R_CHUNK,), jnp.int32),
            pltpu.VMEM((_R_CHUNK,), jnp.int32),
            pltpu.VMEM((_R_CHUNK, W), jnp.float32),
            pltpu.VMEM((_R_CHUNK, W), jnp.float32),
            pltpu.VMEM((_R_CHUNK, CROP_W), jnp.float32),
            pltpu.VMEM((_R_CHUNK, CROP_W), jnp.float32),
            pltpu.SemaphoreType.DMA,
            pltpu.SemaphoreType.DMA,
            pltpu.SemaphoreType.DMA,
            pltpu.SemaphoreType.DMA,
        ],
    )
    def _crop_copy(
        in_hbm, rs_hbm, cs_hbm, out_hbm,
        rs_v, cs_v, idx0, idx1, ib0, ib1, ob0, ob1, gs0, gs1, ws0, ws1,
    ):
        idx = (idx0, idx1)
        ibuf = (ib0, ib1)
        obuf = (ob0, ob1)
        gsem = (gs0, gs1)
        wsem = (ws0, ws1)
        wid = lax.axis_index("s") * _NUM_CORES + lax.axis_index("c")
        pltpu.sync_copy(rs_hbm.at[wid], rs_v)
        pltpu.sync_copy(cs_hbm.at[wid], cs_v)
        iota = lax.iota(jnp.int32, 16)

        def build_idx(par, c):
            plane = c // _CHUNKS_PER_PLANE
            ci = c % _CHUNKS_PER_PLANE
            rs_vec = rs_v[plane]
            r0 = ci * _R_CHUNK
            for k in range(0, _R_CHUNK, 16):
                idx[par][pl.ds(k, 16)] = rs_vec + (r0 + k) + iota

        def start_gather(par):
            pltpu.async_copy(in_hbm.at[idx[par]], ibuf[par], gsem[par])

        def wait_gather(par):
            pltpu.make_async_copy(in_hbm.at[idx[par]], ibuf[par], gsem[par]).wait()

        def write_args(par, c):
            plane = c // _CHUNKS_PER_PLANE
            ci = c % _CHUNKS_PER_PLANE
            dst = (wid * planes_per_w + plane) * CROP_H + ci * _R_CHUNK
            dst = pl.multiple_of(dst, 16)
            return (obuf[par], out_hbm.at[pl.ds(dst, _R_CHUNK)], wsem[par])

        def shift(par, c):
            plane = c // _CHUNKS_PER_PLANE
            cs = jnp.max(cs_v[plane])
            r_vec = cs_v[plane] & 15
            r = cs & 15
            cs16 = pl.multiple_of(cs - r, 16)
            perm = (iota + r_vec) & 15
            mask = iota >= r_vec
            src = ibuf[par]
            dst = obuf[par]

            def _row(i, carry):
                vals = [
                    src[i, pl.ds(cs16 + k, 16)] for k in range(0, CROP_W + 16, 16)
                ]
                for k in range(0, CROP_W, 16):
                    z = jnp.where(mask, vals[k // 16], vals[k // 16 + 1])
                    dst[i, pl.ds(k, 16)] = z.at[perm].get(mode="promise_in_bounds")
                return carry

            lax.fori_loop(0, _R_CHUNK, _row, 0)

        def process(par, c, j):
            wait_gather(par)

            @pl.when(j >= 1)
            def _():
                pltpu.make_async_copy(*write_args(par, c)).wait()

            shift(par, c)
            pltpu.async_copy(*write_args(par, c))

        build_idx(0, 0)
        start_gather(0)

        def body(j, carry):
            c0 = 2 * j
            c1 = c0 + 1
            build_idx(1, c1)
            start_gather(1)
            process(0, c0, j)

            @pl.when(j < n_chunks // 2 - 1)
            def _():
                build_idx(0, c0 + 2)
                start_gather(0)

            process(1, c1, j)
            return carry

        lax.fori_loop(0, n_chunks // 2, body, 0)
        pltpu.make_async_copy(*write_args(0, n_chunks - 2)).wait()
        pltpu.make_async_copy(*write_args(1, n_chunks - 1)).wait()

    out2d = _crop_copy(in2d, rs_rep, cs_rep)
    return out2d.reshape(B * NUM_CROPS, C, CROP_H, CROP_W)


# speedup vs baseline: 3.2918x; 1.1489x over previous
import functools

import jax
import jax.numpy as jnp
from jax import lax
from jax.experimental import pallas as pl
from jax.experimental.pallas import tpu as pltpu
from jax.experimental.pallas import tpu_sc as plsc

CROP_H = 480
CROP_W = 480
NUM_CROPS = 2

_NUM_CORES = 2
_NUM_SUBCORES = 16
_NW = _NUM_CORES * _NUM_SUBCORES

_R_CHUNK = 48
_CHUNKS_PER_PLANE = CROP_H // _R_CHUNK


def _crop_offsets(B, H, W):
    k = jax.random.key(1)
    kh, kw = jax.random.split(k)
    ih = ((H - CROP_H) * jax.random.uniform(kh, (B, NUM_CROPS))).astype(jnp.int32)
    iw = ((W - CROP_W) * jax.random.uniform(kw, (B, NUM_CROPS))).astype(jnp.int32)
    return ih, iw


def kernel(inputs):
    B, C, H, W = inputs.shape
    ih, iw = _crop_offsets(B, H, W)

    P = B * NUM_CROPS * C
    p = jnp.arange(P)
    b_idx = p // (NUM_CROPS * C)
    n_idx = (p // C) % NUM_CROPS
    c_idx = p % C
    row_start = (b_idx * C + c_idx) * H + ih[b_idx, n_idx]
    col_start = iw[b_idx, n_idx]
    planes_per_w = P // _NW
    n_chunks = planes_per_w * _CHUNKS_PER_PLANE
    rs_rep = jnp.broadcast_to(
        row_start.reshape(_NW, planes_per_w, 1).astype(jnp.int32),
        (_NW, planes_per_w, 16),
    )
    cs_rep = jnp.broadcast_to(
        col_start.reshape(_NW, planes_per_w, 1).astype(jnp.int32),
        (_NW, planes_per_w, 16),
    )

    in2d = inputs.reshape(B * C * H, W)
    mesh = plsc.VectorSubcoreMesh(core_axis_name="c", subcore_axis_name="s")

    @functools.partial(
        pl.kernel,
        out_type=jax.ShapeDtypeStruct((P * CROP_H, CROP_W), jnp.float32),
        mesh=mesh,
        compiler_params=pltpu.CompilerParams(needs_layout_passes=False),
        scratch_types=[
            pltpu.VMEM((planes_per_w, 16), jnp.int32),
            pltpu.VMEM((planes_per_w, 16), jnp.int32),
            pltpu.VMEM((_R_CHUNK,), jnp.int32),
            pltpu.VMEM((_R_CHUNK,), jnp.int32),
            pltpu.VMEM((_R_CHUNK, W), jnp.float32),
            pltpu.VMEM((_R_CHUNK, W), jnp.float32),
            pltpu.VMEM((_R_CHUNK, CROP_W), jnp.float32),
            pltpu.VMEM((_R_CHUNK, CROP_W), jnp.float32),
            pltpu.SemaphoreType.DMA,
            pltpu.SemaphoreType.DMA,
            pltpu.SemaphoreType.DMA,
            pltpu.SemaphoreType.DMA,
        ],
    )
    def _crop_copy(
        in_hbm, rs_hbm, cs_hbm, out_hbm,
        rs_v, cs_v, idx0, idx1, ib0, ib1, ob0, ob1, gs0, gs1, ws0, ws1,
    ):
        idx = (idx0, idx1)
        ibuf = (ib0, ib1)
        obuf = (ob0, ob1)
        gsem = (gs0, gs1)
        wsem = (ws0, ws1)
        wid = lax.axis_index("s") * _NUM_CORES + lax.axis_index("c")
        pltpu.sync_copy(rs_hbm.at[wid], rs_v)
        pltpu.sync_copy(cs_hbm.at[wid], cs_v)
        iota = lax.iota(jnp.int32, 16)

        def build_idx(par, c):
            plane = c // _CHUNKS_PER_PLANE
            ci = c % _CHUNKS_PER_PLANE
            rs_vec = rs_v[plane]
            r0 = ci * _R_CHUNK
            for k in range(0, _R_CHUNK, 16):
                idx[par][pl.ds(k, 16)] = rs_vec + (r0 + k) + iota

        def start_gather(par):
            pltpu.async_copy(in_hbm.at[idx[par]], ibuf[par], gsem[par])

        def wait_gather(par):
            pltpu.make_async_copy(in_hbm.at[idx[par]], ibuf[par], gsem[par]).wait()

        def write_args(par, c):
            plane = c // _CHUNKS_PER_PLANE
            ci = c % _CHUNKS_PER_PLANE
            dst = (wid * planes_per_w + plane) * CROP_H + ci * _R_CHUNK
            dst = pl.multiple_of(dst, 16)
            return (obuf[par], out_hbm.at[pl.ds(dst, _R_CHUNK)], wsem[par])

        def shift(par, c):
            plane = c // _CHUNKS_PER_PLANE
            cs = jnp.max(cs_v[plane])
            r_vec = cs_v[plane] & 15
            r = cs & 15
            cs16 = pl.multiple_of(cs - r, 16)
            perm = (iota + r_vec) & 15
            mask = iota >= r_vec
            src = ibuf[par]
            dst = obuf[par]

            @plsc.parallel_loop(0, _R_CHUNK, 1, unroll=2)
            def _row(i):
                vals = [
                    src[i, pl.ds(cs16 + k, 16)] for k in range(0, CROP_W + 16, 16)
                ]
                for k in range(0, CROP_W, 16):
                    z = jnp.where(mask, vals[k // 16], vals[k // 16 + 1])
                    dst[i, pl.ds(k, 16)] = z.at[perm].get(mode="promise_in_bounds")

        def process(par, c, j):
            wait_gather(par)

            @pl.when(j >= 1)
            def _():
                pltpu.make_async_copy(*write_args(par, c)).wait()

            shift(par, c)
            pltpu.async_copy(*write_args(par, c))

        build_idx(0, 0)
        start_gather(0)

        def body(j, carry):
            c0 = 2 * j
            c1 = c0 + 1
            build_idx(1, c1)
            start_gather(1)
            process(0, c0, j)

            @pl.when(j < n_chunks // 2 - 1)
            def _():
                build_idx(0, c0 + 2)
                start_gather(0)

            process(1, c1, j)
            return carry

        lax.fori_loop(0, n_chunks // 2, body, 0)
        pltpu.make_async_copy(*write_args(0, n_chunks - 2)).wait()
        pltpu.make_async_copy(*write_args(1, n_chunks - 1)).wait()

    out2d = _crop_copy(in2d, rs_rep, cs_rep)
    return out2d.reshape(B * NUM_CROPS, C, CROP_H, CROP_W)


# speedup vs baseline: 3.2992x; 1.0023x over previous
import functools

import jax
import jax.numpy as jnp
from jax import lax
from jax.experimental import pallas as pl
from jax.experimental.pallas import tpu as pltpu
from jax.experimental.pallas import tpu_sc as plsc

CROP_H = 480
CROP_W = 480
NUM_CROPS = 2

_NUM_CORES = 2
_NUM_SUBCORES = 16
_NW = _NUM_CORES * _NUM_SUBCORES

_R_CHUNK = 48
_CHUNKS_PER_PLANE = CROP_H // _R_CHUNK


def _crop_offsets(B, H, W):
    k = jax.random.key(1)
    kh, kw = jax.random.split(k)
    ih = ((H - CROP_H) * jax.random.uniform(kh, (B, NUM_CROPS))).astype(jnp.int32)
    iw = ((W - CROP_W) * jax.random.uniform(kw, (B, NUM_CROPS))).astype(jnp.int32)
    return ih, iw


def kernel(inputs):
    B, C, H, W = inputs.shape
    ih, iw = _crop_offsets(B, H, W)

    P = B * NUM_CROPS * C
    p = jnp.arange(P)
    b_idx = p // (NUM_CROPS * C)
    n_idx = (p // C) % NUM_CROPS
    c_idx = p % C
    row_start = (b_idx * C + c_idx) * H + ih[b_idx, n_idx]
    col_start = iw[b_idx, n_idx]
    planes_per_w = P // _NW
    n_chunks = planes_per_w * _CHUNKS_PER_PLANE
    rs_rep = jnp.broadcast_to(
        row_start.reshape(_NW, planes_per_w, 1).astype(jnp.int32),
        (_NW, planes_per_w, 16),
    )
    cs_rep = jnp.broadcast_to(
        col_start.reshape(_NW, planes_per_w, 1).astype(jnp.int32),
        (_NW, planes_per_w, 16),
    )

    in2d = inputs.reshape(B * C * H, W)
    mesh = plsc.VectorSubcoreMesh(core_axis_name="c", subcore_axis_name="s")

    @functools.partial(
        pl.kernel,
        out_type=jax.ShapeDtypeStruct((P * CROP_H, CROP_W), jnp.float32),
        mesh=mesh,
        compiler_params=pltpu.CompilerParams(needs_layout_passes=False),
        scratch_types=[
            pltpu.VMEM((planes_per_w, 16), jnp.int32),
            pltpu.VMEM((planes_per_w, 16), jnp.int32),
            pltpu.VMEM((_R_CHUNK,), jnp.int32),
            pltpu.VMEM((_R_CHUNK,), jnp.int32),
            pltpu.VMEM((_R_CHUNK, W), jnp.float32),
            pltpu.VMEM((_R_CHUNK, W), jnp.float32),
            pltpu.VMEM((_R_CHUNK, CROP_W), jnp.float32),
            pltpu.VMEM((_R_CHUNK, CROP_W), jnp.float32),
            pltpu.SemaphoreType.DMA,
            pltpu.SemaphoreType.DMA,
            pltpu.SemaphoreType.DMA,
            pltpu.SemaphoreType.DMA,
        ],
    )
    def _crop_copy(
        in_hbm, rs_hbm, cs_hbm, out_hbm,
        rs_v, cs_v, idx0, idx1, ib0, ib1, ob0, ob1, gs0, gs1, ws0, ws1,
    ):
        idx = (idx0, idx1)
        ibuf = (ib0, ib1)
        obuf = (ob0, ob1)
        gsem = (gs0, gs1)
        wsem = (ws0, ws1)
        wid = lax.axis_index("s") * _NUM_CORES + lax.axis_index("c")
        pltpu.sync_copy(rs_hbm.at[wid], rs_v)
        pltpu.sync_copy(cs_hbm.at[wid], cs_v)
        iota = lax.iota(jnp.int32, 16)

        def build_idx(par, c):
            plane = c // _CHUNKS_PER_PLANE
            ci = c % _CHUNKS_PER_PLANE
            rs_vec = rs_v[plane]
            r0 = ci * _R_CHUNK
            for k in range(0, _R_CHUNK, 16):
                idx[par][pl.ds(k, 16)] = rs_vec + (r0 + k) + iota

        def start_gather(par):
            pltpu.async_copy(in_hbm.at[idx[par]], ibuf[par], gsem[par])

        def wait_gather(par):
            pltpu.make_async_copy(in_hbm.at[idx[par]], ibuf[par], gsem[par]).wait()

        def write_args(par, c):
            plane = c // _CHUNKS_PER_PLANE
            ci = c % _CHUNKS_PER_PLANE
            dst = (wid * planes_per_w + plane) * CROP_H + ci * _R_CHUNK
            dst = pl.multiple_of(dst, 16)
            return (obuf[par], out_hbm.at[pl.ds(dst, _R_CHUNK)], wsem[par])

        def shift(par, c):
            plane = c // _CHUNKS_PER_PLANE
            cs = jnp.max(cs_v[plane])
            r_vec = cs_v[plane] & 15
            r = cs & 15
            cs16 = pl.multiple_of(cs - r, 16)
            perm = (iota + r_vec) & 15
            mask = iota >= r_vec
            src = ibuf[par]
            dst = obuf[par]

            @plsc.parallel_loop(0, _R_CHUNK, 1, unroll=4)
            def _row(i):
                vals = [
                    src[i, pl.ds(cs16 + k, 16)] for k in range(0, CROP_W + 16, 16)
                ]
                for k in range(0, CROP_W, 16):
                    z = jnp.where(mask, vals[k // 16], vals[k // 16 + 1])
                    dst[i, pl.ds(k, 16)] = z.at[perm].get(mode="promise_in_bounds")

        def process(par, c, j):
            wait_gather(par)

            @pl.when(j >= 1)
            def _():
                pltpu.make_async_copy(*write_args(par, c)).wait()

            shift(par, c)
            pltpu.async_copy(*write_args(par, c))

        build_idx(0, 0)
        start_gather(0)

        def body(j, carry):
            c0 = 2 * j
            c1 = c0 + 1
            build_idx(1, c1)
            start_gather(1)
            process(0, c0, j)

            @pl.when(j < n_chunks // 2 - 1)
            def _():
                build_idx(0, c0 + 2)
                start_gather(0)

            process(1, c1, j)
            return carry

        lax.fori_loop(0, n_chunks // 2, body, 0)
        pltpu.make_async_copy(*write_args(0, n_chunks - 2)).wait()
        pltpu.make_async_copy(*write_args(1, n_chunks - 1)).wait()

    out2d = _crop_copy(in2d, rs_rep, cs_rep)
    return out2d.reshape(B * NUM_CROPS, C, CROP_H, CROP_W)
